# baseline (speedup 1.0000x reference)
import jax
import jax.numpy as jnp
from jax.experimental import pallas as pl

_DELTA = 0.05
_S = 128 * 128
_K = int(_S * 0.3 - 1) + 1
_HI0 = 0x3F800010


def _region_loss_kernel(t_ref, p_ref, out_ref):
    t = t_ref[...]
    p = p_ref[...]
    t0 = t[:, :_S]
    t1 = t[:, _S:2 * _S]
    t2 = t[:, 2 * _S:]
    gray = 0.39 * t0 + 0.5 * t1 + 0.11 * t2
    gi = jax.lax.bitcast_convert_type(gray, jnp.int32)
    b = gray.shape[0]

    def _count(mid):
        return jnp.sum((gi >= mid).astype(jnp.float32), axis=1, keepdims=True)

    def _probe(mid, lo, hi, cl, ch):
        cnt = _count(mid)
        ge = cnt >= _K
        lo = jnp.where(ge, mid, lo)
        cl = jnp.where(ge, cnt, cl)
        hi = jnp.where(ge, hi, mid)
        ch = jnp.where(ge, ch, cnt)
        return lo, hi, cl, ch

    def cond(carry):
        lo, hi, _, _ = carry
        return jnp.any((hi - lo) > 1)

    def body(carry):
        lo, hi, cl, ch = carry
        vlo = jax.lax.bitcast_convert_type(lo, jnp.float32)
        vhi = jax.lax.bitcast_convert_type(hi, jnp.float32)
        frac = (cl - _K) / jnp.maximum(cl - ch, 1.0)
        vm = vlo + (vhi - vlo) * frac
        m1 = jax.lax.bitcast_convert_type(vm, jnp.int32)
        m1 = jnp.clip(m1, lo + 1, hi - 1)
        lo, hi, cl, ch = _probe(m1, lo, hi, cl, ch)
        m2 = lo + ((hi - lo) >> 1)
        lo, hi, cl, ch = _probe(m2, lo, hi, cl, ch)
        return lo, hi, cl, ch

    lo0 = jnp.zeros((b, 1), jnp.int32)
    hi0 = jnp.full((b, 1), _HI0, jnp.int32)
    cl0 = jnp.full((b, 1), float(_S), jnp.float32)
    ch0 = jnp.zeros((b, 1), jnp.float32)
    lo, _, _, _ = jax.lax.while_loop(cond, body, (lo0, hi0, cl0, ch0))
    yu = jax.lax.bitcast_convert_type(lo, jnp.float32)

    d = jnp.abs(t - p)
    f = jnp.where(d < _DELTA, 0.5 * d * d, _DELTA * d - 0.5 * _DELTA * _DELTA)
    m = (gray >= yu).astype(jnp.float32)
    fm = (f[:, :_S] + f[:, _S:2 * _S] + f[:, 2 * _S:]) * m
    ones3s = jnp.ones((3 * _S, 8), jnp.float32)
    sf = jax.lax.dot_general(f, ones3s, (((1,), (0,)), ((), ())),
                             preferred_element_type=jnp.float32)[:, :1]
    sfm = jax.lax.dot_general(fm, ones3s[:_S], (((1,), (0,)), ((), ())),
                              preferred_element_type=jnp.float32)[:, :1]
    out_ref[...] = jnp.sum(sf + 3.0 * sfm).reshape(1, 1, 1)


def kernel(pred, target):
    n, c, h, w = pred.shape
    s = h * w
    pr = pred.reshape(n, c * s)
    tr = target.reshape(n, c * s)
    blk = 32
    grid = n // blk
    partial = pl.pallas_call(
        _region_loss_kernel,
        grid=(grid,),
        in_specs=[
            pl.BlockSpec((blk, c * s), lambda i: (i, 0)),
            pl.BlockSpec((blk, c * s), lambda i: (i, 0)),
        ],
        out_specs=pl.BlockSpec((1, 1, 1), lambda i: (i, 0, 0)),
        out_shape=jax.ShapeDtypeStruct((grid, 1, 1), jnp.float32),
    )(tr, pr)
    return jnp.sum(partial) * (1.0 / (n * c * s))
